# baseline (speedup 1.0000x reference)
import functools

import jax
import jax.numpy as jnp
from jax import lax
from jax.experimental import pallas as pl
from jax.experimental.pallas import tpu as pltpu
from jax.experimental.pallas import tpu_sc as plsc

N_NODES = 325
F = 16
H1 = 128
H2 = 64
NCLS = 64
WIN = 12
WOUT = 3
B = 32
N = B * WIN * N_NODES
E = 4 * N
ROWS = B * WIN
DIN = N_NODES * F

ROW_BLK = 4160
NB = N // 8
PBLK = 3120

NC = 2
NS = 16
NW = NC * NS
CHUNK = 96
NBLK = 5
NCHUNK = E // (NS * NBLK * CHUNK)
NBUF = 13
HALF = N // 2
PAD = 64
ACC_ROWS = HALF + PAD
ZPT = ACC_ROWS // NS
OPT = HALF // 8
_SC_MESH = plsc.VectorSubcoreMesh(core_axis_name="c", subcore_axis_name="s")


def _core_idx(didx, j, c):
    lo = c * HALF
    for v in range(CHUNK // 16):
        d = didx[j, pl.ds(16 * v, 16)]
        t = d - lo
        ok = (t >= 0) & (t < HALF)
        didx[j, pl.ds(16 * v, 16)] = jnp.where(
            ok, t, jnp.bitwise_and(d, PAD - 1) + HALF)


@functools.partial(
    pl.kernel,
    out_type=jax.ShapeDtypeStruct((N, F), jnp.float32),
    mesh=_SC_MESH,
    scratch_types=[
        pltpu.VMEM((NCHUNK, CHUNK), jnp.int32),
        pltpu.VMEM((CHUNK, F), jnp.float32),
        pltpu.VMEM_SHARED((ACC_ROWS, F), jnp.float32),
        pltpu.SemaphoreType.DMA,
    ],
    compiler_params=pltpu.CompilerParams(use_tc_tiling_on_sc=False),
)
def _sc_deg(dst_hbm, out_hbm, didx, rows, acc, sem):
    c = lax.axis_index("c")
    s = lax.axis_index("s")

    @pl.loop(0, CHUNK)
    def _(i):
        rows[i] = jnp.zeros((F,), jnp.float32)

    @pl.loop(0, ZPT // 64)
    def _(r):
        pltpu.sync_copy(rows.at[pl.ds(0, 64)],
                        acc.at[pl.ds(s * ZPT + r * 64, 64)])

    @pl.loop(0, CHUNK)
    def _(i):
        rows[i] = jnp.ones((F,), jnp.float32)

    plsc.subcore_barrier()

    @pl.loop(0, NBLK)
    def _(kb):
        pltpu.sync_copy(dst_hbm.at[NBLK * s + kb], didx)

        @pl.loop(0, NCHUNK // 13)
        def _(gq):
            @pl.loop(0, 13)
            def _(u):
                j = gq * 13 + u
                _core_idx(didx, j, c)
                pltpu.async_copy(rows, acc.at[didx.at[j]], sem, add=True)

            @pl.loop(0, 13)
            def _(u):
                pltpu.make_async_copy(rows, acc.at[didx.at[0]], sem).wait()

    plsc.subcore_barrier()

    @pl.when(s < 8)
    def _():
        pltpu.sync_copy(acc.at[pl.ds(s * OPT, OPT)],
                        out_hbm.at[pl.ds(c * HALF + s * OPT, OPT)])


@functools.partial(
    pl.kernel,
    out_type=jax.ShapeDtypeStruct((N, F), jnp.float32),
    mesh=_SC_MESH,
    scratch_types=[
        pltpu.VMEM((NCHUNK, CHUNK), jnp.int32),
        pltpu.VMEM((NCHUNK, CHUNK), jnp.int32),
        pltpu.VMEM((NBUF, CHUNK, F), jnp.float32),
        pltpu.VMEM_SHARED((ACC_ROWS, F), jnp.float32),
        [pltpu.SemaphoreType.DMA] * NBUF,
        [pltpu.SemaphoreType.DMA] * NBUF,
    ],
    compiler_params=pltpu.CompilerParams(use_tc_tiling_on_sc=False),
)
def _sc_agg(vals_hbm, src_hbm, dst_hbm, out_hbm, sidx, didx, rows, acc,
            gs, ss):
    c = lax.axis_index("c")
    s = lax.axis_index("s")

    @pl.loop(0, CHUNK)
    def _(i):
        rows[0, i] = jnp.zeros((F,), jnp.float32)

    @pl.loop(0, ZPT // 64)
    def _(r):
        pltpu.sync_copy(rows.at[0, pl.ds(0, 64)],
                        acc.at[pl.ds(s * ZPT + r * 64, 64)])

    plsc.subcore_barrier()

    @pl.loop(0, NBLK)
    def _(kb):
        pltpu.sync_copy(src_hbm.at[NBLK * s + kb], sidx)
        pltpu.sync_copy(dst_hbm.at[NBLK * s + kb], didx)
        for b in range(NBUF):
            pltpu.async_copy(vals_hbm.at[sidx.at[b]], rows.at[b], gs[b])

        @pl.loop(0, NCHUNK // NBUF)
        def _(p):
            j = NBUF * p
            for b in range(NBUF):
                _core_idx(didx, j + b, c)
                pltpu.make_async_copy(vals_hbm.at[sidx.at[j + b]],
                                      rows.at[b], gs[b]).wait()
                pltpu.async_copy(rows.at[b], acc.at[didx.at[j + b]], ss[b],
                                 add=True)

            @pl.when(p < NCHUNK // NBUF - 1)
            def _():
                for b in range(NBUF):
                    pltpu.make_async_copy(rows.at[b], acc.at[didx.at[0]],
                                          ss[b]).wait()
                    pltpu.async_copy(vals_hbm.at[sidx.at[j + NBUF + b]],
                                     rows.at[b], gs[b])

        for b in range(NBUF):
            pltpu.make_async_copy(rows.at[b], acc.at[didx.at[0]], ss[b]).wait()

    plsc.subcore_barrier()

    @pl.when(s < 8)
    def _():
        pltpu.sync_copy(acc.at[pl.ds(s * OPT, OPT)],
                        out_hbm.at[pl.ds(c * HALF + s * OPT, OPT)])


def _scale_body(deg_ref, x_ref, xs_ref, dinv_ref):
    dinv = jax.lax.rsqrt(deg_ref[...] + 1.0)
    dinv_ref[...] = dinv
    xs_ref[...] = x_ref[...] * dinv


def _scale(degp, xp):
    grid = (NB // PBLK,)
    bs = pl.BlockSpec((PBLK, 128), lambda i: (i, 0))
    return pl.pallas_call(
        _scale_body,
        grid=grid,
        in_specs=[bs, bs],
        out_specs=[bs, bs],
        out_shape=[jax.ShapeDtypeStruct((NB, 128), jnp.float32)] * 2,
    )(degp, xp)


def _dense1_body(agg_ref, xs_ref, dinv_ref, w1_ref, b1_ref, w2_ref,
                 ys_ref):
    dinv = dinv_ref[...]
    z1 = (agg_ref[...] + xs_ref[...]) * dinv
    h1 = jnp.maximum(
        jnp.dot(z1, w1_ref[...], preferred_element_type=jnp.float32)
        + b1_ref[...], 0.0)
    y = jnp.dot(h1, w2_ref[...], preferred_element_type=jnp.float32)
    ys_ref[...] = y * dinv


def _dense1(agg, xs, dinv, W1, b1, W2):
    eye8 = jnp.eye(8, dtype=jnp.float32)
    w1k = jnp.kron(eye8, W1)
    w2k = jnp.kron(eye8, W2)
    b1k = jnp.tile(b1, 8).reshape(1, 8 * H1)
    grid = (NB // PBLK,)
    bs = pl.BlockSpec((PBLK, 128), lambda i: (i, 0))
    return pl.pallas_call(
        _dense1_body,
        grid=grid,
        in_specs=[
            bs, bs, bs,
            pl.BlockSpec((128, 8 * H1), lambda i: (0, 0)),
            pl.BlockSpec((1, 8 * H1), lambda i: (0, 0)),
            pl.BlockSpec((8 * H1, 128), lambda i: (0, 0)),
        ],
        out_specs=bs,
        out_shape=jax.ShapeDtypeStruct((NB, 128), jnp.float32),
    )(agg, xs, dinv, w1k, b1k, w2k)


def _dense2_body(agg_ref, ys_ref, dinv_ref, b2_ref, h2_ref):
    z2 = (agg_ref[...] + ys_ref[...]) * dinv_ref[...]
    h2_ref[...] = jnp.maximum(z2 + b2_ref[...], 0.0)


def _dense2(agg, ys, dinv, b2):
    b2k = jnp.tile(b2, 8).reshape(1, 128)
    grid = (NB // PBLK,)
    bs = pl.BlockSpec((PBLK, 128), lambda i: (i, 0))
    return pl.pallas_call(
        _dense2_body,
        grid=grid,
        in_specs=[bs, bs, bs, pl.BlockSpec((1, 128), lambda i: (0, 0))],
        out_specs=bs,
        out_shape=jax.ShapeDtypeStruct((NB, 128), jnp.float32),
    )(agg, ys, dinv, b2k)


def _lstm_body(xt_ref, wih0_ref, bg0_ref, whh0_ref, wih1_ref, bg1_ref,
               whh1_ref, wf1_ref, bf1_ref, wf2_ref, bf2_ref, out_ref, g_ref):
    g = jnp.dot(xt_ref[...], wih0_ref[...], preferred_element_type=jnp.float32)
    g_ref[...] = g.reshape(B, WIN, 4 * H1) + bg0_ref[...]

    def step(t, carry):
        h0, c0, h1, c1 = carry
        g0 = g_ref[:, t, :] + jnp.dot(h0, whh0_ref[...],
                                      preferred_element_type=jnp.float32)
        i0 = jax.nn.sigmoid(g0[:, 0:H1])
        f0 = jax.nn.sigmoid(g0[:, H1:2 * H1])
        z0 = jnp.tanh(g0[:, 2 * H1:3 * H1])
        o0 = jax.nn.sigmoid(g0[:, 3 * H1:])
        c0 = f0 * c0 + i0 * z0
        h0 = o0 * jnp.tanh(c0)

        g1 = (jnp.dot(h0, wih1_ref[...], preferred_element_type=jnp.float32)
              + jnp.dot(h1, whh1_ref[...], preferred_element_type=jnp.float32)
              + bg1_ref[...])
        i1 = jax.nn.sigmoid(g1[:, 0:H1])
        f1 = jax.nn.sigmoid(g1[:, H1:2 * H1])
        z1 = jnp.tanh(g1[:, 2 * H1:3 * H1])
        o1 = jax.nn.sigmoid(g1[:, 3 * H1:])
        c1 = f1 * c1 + i1 * z1
        h1 = o1 * jnp.tanh(c1)

        @pl.when(t >= WIN - WOUT)
        def _():
            ff = jnp.maximum(
                jnp.dot(h1, wf1_ref[...], preferred_element_type=jnp.float32)
                + bf1_ref[...], 0.0)
            out_ref[t - (WIN - WOUT)] = (
                jnp.dot(ff, wf2_ref[...], preferred_element_type=jnp.float32)
                + bf2_ref[...])

        return h0, c0, h1, c1

    zero = jnp.zeros((B, H1), jnp.float32)
    jax.lax.fori_loop(0, WIN, step, (zero, zero, zero, zero))


def _lstm_head(xt, Wih0T, bg0, Whh0T, Wih1T, bg1, Whh1T, Wf1, bf1, Wf2, bf2):
    return pl.pallas_call(
        _lstm_body,
        in_specs=[pl.BlockSpec(memory_space=pltpu.VMEM) for _ in range(11)],
        out_specs=pl.BlockSpec(memory_space=pltpu.VMEM),
        out_shape=jax.ShapeDtypeStruct((WOUT, B, NCLS), jnp.float32),
        scratch_shapes=[pltpu.VMEM((B, WIN, 4 * H1), jnp.float32)],
    )(xt, Wih0T, bg0.reshape(1, 1, 4 * H1), Whh0T, Wih1T,
      bg1.reshape(1, 4 * H1), Whh1T, Wf1, bf1.reshape(1, H2), Wf2,
      bf2.reshape(1, NCLS))


def kernel(x, edge_index, W1, b1, W2, b2, W_ih0, W_hh0, b_ih0, b_hh0,
           W_ih1, W_hh1, b_ih1, b_hh1, Wf1, bf1, Wf2, bf2):
    src3d = edge_index[0].reshape(NS * NBLK, NCHUNK, CHUNK)
    dst3d = edge_index[1].reshape(NS * NBLK, NCHUNK, CHUNK)

    deg16 = _sc_deg(dst3d)
    xs, dinv = _scale(deg16.reshape(NB, 128), x.reshape(NB, 128))

    agg1 = _sc_agg(xs.reshape(N, F), src3d, dst3d)
    ys = _dense1(agg1.reshape(NB, 128), xs, dinv, W1, b1, W2)

    agg2 = _sc_agg(ys.reshape(N, F), src3d, dst3d)
    h2 = _dense2(agg2.reshape(NB, 128), ys, dinv, b2)

    xt = h2.reshape(B * WIN, DIN)

    out = _lstm_head(
        xt, W_ih0.T, b_ih0 + b_hh0, W_hh0.T, W_ih1.T, b_ih1 + b_hh1,
        W_hh1.T, Wf1, bf1, Wf2, bf2)
    return out.transpose(1, 0, 2).reshape(B * WOUT, NCLS)


# speedup vs baseline: 1.0075x; 1.0075x over previous
import functools

import jax
import jax.numpy as jnp
from jax import lax
from jax.experimental import pallas as pl
from jax.experimental.pallas import tpu as pltpu
from jax.experimental.pallas import tpu_sc as plsc

N_NODES = 325
F = 16
H1 = 128
H2 = 64
NCLS = 64
WIN = 12
WOUT = 3
B = 32
N = B * WIN * N_NODES
E = 4 * N
ROWS = B * WIN
DIN = N_NODES * F

ROW_BLK = 4160
NB = N // 8
PBLK = 3120

NC = 2
NS = 16
NW = NC * NS
CHUNK = 96
NBLK = 5
NCHUNK = E // (NS * NBLK * CHUNK)
NBUF = 13
HALF = N // 2
PAD = 64
ACC_ROWS = HALF + PAD
ZPT = ACC_ROWS // NS
OPT = HALF // 8
_SC_MESH = plsc.VectorSubcoreMesh(core_axis_name="c", subcore_axis_name="s")


def _core_idx(didx, j, c):
    lo = c * HALF
    for v in range(CHUNK // 16):
        d = didx[j, pl.ds(16 * v, 16)]
        t = d - lo
        ok = (t >= 0) & (t < HALF)
        didx[j, pl.ds(16 * v, 16)] = jnp.where(
            ok, t, jnp.bitwise_and(d, PAD - 1) + HALF)


@functools.partial(
    pl.kernel,
    out_type=jax.ShapeDtypeStruct((N, F), jnp.float32),
    mesh=_SC_MESH,
    scratch_types=[
        pltpu.VMEM((NCHUNK, CHUNK), jnp.int32),
        pltpu.VMEM((CHUNK, F), jnp.float32),
        pltpu.VMEM_SHARED((ACC_ROWS, F), jnp.float32),
        pltpu.SemaphoreType.DMA,
    ],
    compiler_params=pltpu.CompilerParams(use_tc_tiling_on_sc=False),
)
def _sc_deg(dst_hbm, out_hbm, didx, rows, acc, sem):
    c = lax.axis_index("c")
    s = lax.axis_index("s")

    @pl.loop(0, CHUNK)
    def _(i):
        rows[i] = jnp.zeros((F,), jnp.float32)

    @pl.loop(0, ZPT // 64)
    def _(r):
        pltpu.sync_copy(rows.at[pl.ds(0, 64)],
                        acc.at[pl.ds(s * ZPT + r * 64, 64)])

    @pl.loop(0, CHUNK)
    def _(i):
        rows[i] = jnp.ones((F,), jnp.float32)

    plsc.subcore_barrier()

    @pl.loop(0, NBLK)
    def _(kb):
        pltpu.sync_copy(dst_hbm.at[NBLK * s + kb], didx)

        @pl.loop(0, NCHUNK // 13)
        def _(gq):
            @pl.loop(0, 13)
            def _(u):
                j = gq * 13 + u
                _core_idx(didx, j, c)
                pltpu.async_copy(rows, acc.at[didx.at[j]], sem, add=True)

            @pl.loop(0, 13)
            def _(u):
                pltpu.make_async_copy(rows, acc.at[didx.at[0]], sem).wait()

    plsc.subcore_barrier()

    @pl.when(s < 8)
    def _():
        pltpu.sync_copy(acc.at[pl.ds(s * OPT, OPT)],
                        out_hbm.at[pl.ds(c * HALF + s * OPT, OPT)])


@functools.partial(
    pl.kernel,
    out_type=jax.ShapeDtypeStruct((N, F), jnp.float32),
    mesh=_SC_MESH,
    scratch_types=[
        pltpu.VMEM((NCHUNK, CHUNK), jnp.int32),
        pltpu.VMEM((NCHUNK, CHUNK), jnp.int32),
        pltpu.VMEM((NBUF, CHUNK, F), jnp.float32),
        pltpu.VMEM_SHARED((ACC_ROWS, F), jnp.float32),
        [pltpu.SemaphoreType.DMA] * NBUF,
        [pltpu.SemaphoreType.DMA] * NBUF,
    ],
    compiler_params=pltpu.CompilerParams(use_tc_tiling_on_sc=False),
)
def _sc_agg(vals_hbm, src_hbm, dst_hbm, out_hbm, sidx, didx, rows, acc,
            gs, ss):
    c = lax.axis_index("c")
    s = lax.axis_index("s")

    @pl.loop(0, CHUNK)
    def _(i):
        rows[0, i] = jnp.zeros((F,), jnp.float32)

    @pl.loop(0, ZPT // 64)
    def _(r):
        pltpu.sync_copy(rows.at[0, pl.ds(0, 64)],
                        acc.at[pl.ds(s * ZPT + r * 64, 64)])

    plsc.subcore_barrier()

    @pl.loop(0, NBLK)
    def _(kb):
        pltpu.sync_copy(src_hbm.at[NBLK * s + kb], sidx)
        pltpu.sync_copy(dst_hbm.at[NBLK * s + kb], didx)
        for b in range(NBUF):
            pltpu.async_copy(vals_hbm.at[sidx.at[b]], rows.at[b], gs[b])

        @pl.loop(0, NCHUNK // NBUF)
        def _(p):
            j = NBUF * p
            for b in range(NBUF):
                _core_idx(didx, j + b, c)
                pltpu.make_async_copy(vals_hbm.at[sidx.at[j + b]],
                                      rows.at[b], gs[b]).wait()
                pltpu.async_copy(rows.at[b], acc.at[didx.at[j + b]], ss[b],
                                 add=True)

            @pl.when(p < NCHUNK // NBUF - 1)
            def _():
                for b in range(NBUF):
                    pltpu.make_async_copy(rows.at[b], acc.at[didx.at[0]],
                                          ss[b]).wait()
                    pltpu.async_copy(vals_hbm.at[sidx.at[j + NBUF + b]],
                                     rows.at[b], gs[b])

        for b in range(NBUF):
            pltpu.make_async_copy(rows.at[b], acc.at[didx.at[0]], ss[b]).wait()

    plsc.subcore_barrier()

    @pl.when(s < 8)
    def _():
        pltpu.sync_copy(acc.at[pl.ds(s * OPT, OPT)],
                        out_hbm.at[pl.ds(c * HALF + s * OPT, OPT)])


def _edges_body(ei_ref, src_ref, dst_ref):
    src_ref[...] = ei_ref[0].reshape(E // 128, 128)
    dst_ref[...] = ei_ref[1].reshape(E // 128, 128)


def _edges(edge_index):
    return pl.pallas_call(
        _edges_body,
        in_specs=[pl.BlockSpec((2, E), lambda: (0, 0))],
        out_specs=[pl.BlockSpec((E // 128, 128), lambda: (0, 0)),
                   pl.BlockSpec((E // 128, 128), lambda: (0, 0))],
        out_shape=[jax.ShapeDtypeStruct((E // 128, 128), jnp.int32)] * 2,
    )(edge_index)


def _scale_body(deg_ref, x_ref, xs_ref):
    xs_ref[...] = x_ref[...] * jax.lax.rsqrt(deg_ref[...] + 1.0)


def _scale(degp, xp):
    grid = (NB // PBLK,)
    bs = pl.BlockSpec((PBLK, 128), lambda i: (i, 0))
    return pl.pallas_call(
        _scale_body,
        grid=grid,
        in_specs=[bs, bs],
        out_specs=bs,
        out_shape=jax.ShapeDtypeStruct((NB, 128), jnp.float32),
    )(degp, xp)


def _dense1_body(agg_ref, xs_ref, deg_ref, w1_ref, b1_ref, w2_ref,
                 ys_ref):
    dinv = jax.lax.rsqrt(deg_ref[...] + 1.0)
    z1 = (agg_ref[...] + xs_ref[...]) * dinv
    h1 = jnp.maximum(
        jnp.dot(z1, w1_ref[...], preferred_element_type=jnp.float32)
        + b1_ref[...], 0.0)
    y = jnp.dot(h1, w2_ref[...], preferred_element_type=jnp.float32)
    ys_ref[...] = y * dinv


def _dense1(agg, xs, degp, W1, b1, W2):
    eye8 = jnp.eye(8, dtype=jnp.float32)
    w1k = jnp.kron(eye8, W1)
    w2k = jnp.kron(eye8, W2)
    b1k = jnp.tile(b1, 8).reshape(1, 8 * H1)
    grid = (NB // PBLK,)
    bs = pl.BlockSpec((PBLK, 128), lambda i: (i, 0))
    return pl.pallas_call(
        _dense1_body,
        grid=grid,
        in_specs=[
            bs, bs, bs,
            pl.BlockSpec((128, 8 * H1), lambda i: (0, 0)),
            pl.BlockSpec((1, 8 * H1), lambda i: (0, 0)),
            pl.BlockSpec((8 * H1, 128), lambda i: (0, 0)),
        ],
        out_specs=bs,
        out_shape=jax.ShapeDtypeStruct((NB, 128), jnp.float32),
    )(agg, xs, degp, w1k, b1k, w2k)


def _dense2_body(agg_ref, ys_ref, deg_ref, b2_ref, h2_ref):
    z2 = ((agg_ref[...] + ys_ref[...])
          * jax.lax.rsqrt(deg_ref[...] + 1.0))
    h2_ref[...] = jnp.maximum(z2 + b2_ref[...], 0.0)


def _dense2(agg, ys, degp, b2):
    b2k = jnp.tile(b2, 8).reshape(1, 128)
    grid = (NB // PBLK,)
    bs = pl.BlockSpec((PBLK, 128), lambda i: (i, 0))
    return pl.pallas_call(
        _dense2_body,
        grid=grid,
        in_specs=[bs, bs, bs, pl.BlockSpec((1, 128), lambda i: (0, 0))],
        out_specs=bs,
        out_shape=jax.ShapeDtypeStruct((NB, 128), jnp.float32),
    )(agg, ys, degp, b2k)


def _lstm_body(xt_ref, wih0_ref, bg0_ref, whh0_ref, wih1_ref, bg1_ref,
               whh1_ref, wf1_ref, bf1_ref, wf2_ref, bf2_ref, out_ref, g_ref):
    g = jnp.dot(xt_ref[...], wih0_ref[...], preferred_element_type=jnp.float32)
    g_ref[...] = g.reshape(B, WIN, 4 * H1) + bg0_ref[...]

    def step(t, carry):
        h0, c0, h1, c1 = carry
        g0 = g_ref[:, t, :] + jnp.dot(h0, whh0_ref[...],
                                      preferred_element_type=jnp.float32)
        i0 = jax.nn.sigmoid(g0[:, 0:H1])
        f0 = jax.nn.sigmoid(g0[:, H1:2 * H1])
        z0 = jnp.tanh(g0[:, 2 * H1:3 * H1])
        o0 = jax.nn.sigmoid(g0[:, 3 * H1:])
        c0 = f0 * c0 + i0 * z0
        h0 = o0 * jnp.tanh(c0)

        g1 = (jnp.dot(h0, wih1_ref[...], preferred_element_type=jnp.float32)
              + jnp.dot(h1, whh1_ref[...], preferred_element_type=jnp.float32)
              + bg1_ref[...])
        i1 = jax.nn.sigmoid(g1[:, 0:H1])
        f1 = jax.nn.sigmoid(g1[:, H1:2 * H1])
        z1 = jnp.tanh(g1[:, 2 * H1:3 * H1])
        o1 = jax.nn.sigmoid(g1[:, 3 * H1:])
        c1 = f1 * c1 + i1 * z1
        h1 = o1 * jnp.tanh(c1)

        @pl.when(t >= WIN - WOUT)
        def _():
            ff = jnp.maximum(
                jnp.dot(h1, wf1_ref[...], preferred_element_type=jnp.float32)
                + bf1_ref[...], 0.0)
            out_ref[t - (WIN - WOUT)] = (
                jnp.dot(ff, wf2_ref[...], preferred_element_type=jnp.float32)
                + bf2_ref[...])

        return h0, c0, h1, c1

    zero = jnp.zeros((B, H1), jnp.float32)
    jax.lax.fori_loop(0, WIN, step, (zero, zero, zero, zero))


def _lstm_head(xt, Wih0T, bg0, Whh0T, Wih1T, bg1, Whh1T, Wf1, bf1, Wf2, bf2):
    return pl.pallas_call(
        _lstm_body,
        in_specs=[pl.BlockSpec(memory_space=pltpu.VMEM) for _ in range(11)],
        out_specs=pl.BlockSpec(memory_space=pltpu.VMEM),
        out_shape=jax.ShapeDtypeStruct((WOUT, B, NCLS), jnp.float32),
        scratch_shapes=[pltpu.VMEM((B, WIN, 4 * H1), jnp.float32)],
    )(xt, Wih0T, bg0.reshape(1, 1, 4 * H1), Whh0T, Wih1T,
      bg1.reshape(1, 4 * H1), Whh1T, Wf1, bf1.reshape(1, H2), Wf2,
      bf2.reshape(1, NCLS))


def kernel(x, edge_index, W1, b1, W2, b2, W_ih0, W_hh0, b_ih0, b_hh0,
           W_ih1, W_hh1, b_ih1, b_hh1, Wf1, bf1, Wf2, bf2):
    srcE, dstE = _edges(edge_index)
    src3d = srcE.reshape(NS * NBLK, NCHUNK, CHUNK)
    dst3d = dstE.reshape(NS * NBLK, NCHUNK, CHUNK)

    deg16 = _sc_deg(dst3d)
    degp = deg16.reshape(NB, 128)
    xs = _scale(degp, x.reshape(NB, 128))

    agg1 = _sc_agg(xs.reshape(N, F), src3d, dst3d)
    ys = _dense1(agg1.reshape(NB, 128), xs, degp, W1, b1, W2)

    agg2 = _sc_agg(ys.reshape(N, F), src3d, dst3d)
    h2 = _dense2(agg2.reshape(NB, 128), ys, degp, b2)

    xt = h2.reshape(B * WIN, DIN)

    out = _lstm_head(
        xt, W_ih0.T, b_ih0 + b_hh0, W_hh0.T, W_ih1.T, b_ih1 + b_hh1,
        W_hh1.T, Wf1, bf1, Wf2, bf2)
    return out.transpose(1, 0, 2).reshape(B * WOUT, NCLS)
